# baseline (speedup 1.0000x reference)
import functools

import jax
import jax.numpy as jnp
from jax import lax
from jax.experimental import pallas as pl
from jax.experimental.pallas import tpu as pltpu
from jax.experimental.pallas import tpu_sc as plsc

E = 8
NGROUP = 4
GSIZE = E // NGROUP
H = 768
DFF = 384
SCALE = 2.5
N = 2048

NWORK = 32
TPW = N // NWORK

_RHS_T = (((1,), (1,)), ((), ()))


def _gate_prep_kernel(x_ref, wgate_ref, bias_ref, sc_ref, sb_ref, xb_ref):
    x = x_ref[...]
    logits = lax.dot_general(
        x, wgate_ref[...], _RHS_T,
        precision=lax.Precision.DEFAULT,
        preferred_element_type=jnp.float32)
    st = jax.nn.sigmoid(logits.T)
    sc_ref[...] = st
    sb_ref[...] = st + bias_ref[...]
    xb_ref[...] = x.astype(jnp.bfloat16)


def _route16(sc, sb):
    neg = jnp.float32(-1e30)
    one = jnp.ones((16,), jnp.float32)
    zero = jnp.zeros((16,), jnp.float32)

    def f(b):
        return jnp.where(b, one, zero)

    g = [sb[2 * j] + sb[2 * j + 1] for j in range(NGROUP)]
    grank = []
    for j in range(NGROUP):
        rj = zero
        for k in range(NGROUP):
            if k == j:
                continue
            rj = rj + f(g[k] > g[j])
            if k < j:
                rj = rj + f(g[k] == g[j])
        grank.append(rj)
    masked = [jnp.where(grank[e // GSIZE] < 1.5, sb[e], neg) for e in range(E)]
    m1 = masked[0]
    for e in range(1, E):
        m1 = jnp.maximum(m1, masked[e])
    top1 = []
    seen = zero
    for e in range(E):
        hit = f(masked[e] == m1)
        t = hit * (one - seen)
        top1.append(t)
        seen = seen + t
    rest = [jnp.where(top1[e] > 0.5, neg, masked[e]) for e in range(E)]
    m2 = rest[0]
    for e in range(1, E):
        m2 = jnp.maximum(m2, rest[e])
    top2 = []
    seen = zero
    for e in range(E):
        hit = f(rest[e] == m2)
        t = hit * (one - seen)
        top2.append(t)
        seen = seen + t
    w1raw = zero
    w2raw = zero
    for e in range(E):
        w1raw = w1raw + top1[e] * sc[e]
        w2raw = w2raw + top2[e] * sc[e]
    inv = SCALE / (w1raw + w2raw + 1e-20)
    return [(top1[e] * w1raw + top2[e] * w2raw) * inv for e in range(E)]


def _sc_routing_kernel(sc_hbm, sb_hbm, comb_hbm, sc_v, sb_v, out_v):
    wid = lax.axis_index("s") * 2 + lax.axis_index("c")
    for e in range(E):
        pltpu.sync_copy(sc_hbm.at[pl.ds(e * N + wid * TPW, TPW)],
                        sc_v.at[pl.ds(e * TPW, TPW)])
        pltpu.sync_copy(sb_hbm.at[pl.ds(e * N + wid * TPW, TPW)],
                        sb_v.at[pl.ds(e * TPW, TPW)])
    for t in range(TPW // 16):
        sc = [sc_v[pl.ds(e * TPW + t * 16, 16)] for e in range(E)]
        sb = [sb_v[pl.ds(e * TPW + t * 16, 16)] for e in range(E)]
        comb = _route16(sc, sb)
        for e in range(E):
            out_v[pl.ds(e * TPW + t * 16, 16)] = comb[e]
    for e in range(E):
        pltpu.sync_copy(out_v.at[pl.ds(e * TPW, TPW)],
                        comb_hbm.at[pl.ds(e * N + wid * TPW, TPW)])


def _moe_kernel(xb_ref, combEN_ref, wg_hbm, wu_hbm, wd_hbm,
                wsg_hbm, wsu_hbm, wsd_hbm, out_ref,
                g_stg, u_stg, d_stg, comb_ref, act_ref, wd2_ref, sems):
    def copies(e, b):
        if e < E:
            return (pltpu.make_async_copy(wg_hbm.at[e], g_stg.at[b], sems.at[b]),
                    pltpu.make_async_copy(wu_hbm.at[e], u_stg.at[b], sems.at[b]),
                    pltpu.make_async_copy(wd_hbm.at[e], d_stg.at[b], sems.at[b]))
        return (pltpu.make_async_copy(wsg_hbm, g_stg.at[b], sems.at[b]),
                pltpu.make_async_copy(wsu_hbm, u_stg.at[b], sems.at[b]),
                pltpu.make_async_copy(wsd_hbm, d_stg.at[b], sems.at[b]))

    for c in copies(0, 0):
        c.start()

    comb_ref[...] = combEN_ref[...].T
    for e in range(E + 1):
        b = e % 2
        if e < E:
            for c in copies(e + 1, 1 - b):
                c.start()
        for c in copies(e, b):
            c.wait()
        xb = xb_ref[...]
        wgb = g_stg[b].astype(jnp.bfloat16)
        wub = u_stg[b].astype(jnp.bfloat16)
        wd2_ref[:, e * DFF:(e + 1) * DFF] = d_stg[b].astype(jnp.bfloat16)
        gm = lax.dot_general(xb, wgb, _RHS_T,
                             preferred_element_type=jnp.float32)
        um = lax.dot_general(xb, wub, _RHS_T,
                             preferred_element_type=jnp.float32)
        act = gm * jax.nn.sigmoid(gm) * um
        if e < E:
            act = act * comb_ref[:, e:e + 1]
        act_ref[:, e * DFF:(e + 1) * DFF] = act.astype(jnp.bfloat16)
    out_ref[...] = lax.dot_general(act_ref[...], wd2_ref[...], _RHS_T,
                                   preferred_element_type=jnp.float32)


def kernel(hidden_states, image_mask, audio_mask, Wg, expert_bias,
           W_gate, W_up, W_down, Ws_gate, Ws_up, Ws_down):
    orig_shape = hidden_states.shape
    x = hidden_states.reshape(-1, H)
    biasT = expert_bias.reshape(E, 1)

    scm, sbm, xb = pl.pallas_call(
        _gate_prep_kernel,
        in_specs=[
            pl.BlockSpec((N, H), lambda: (0, 0)),
            pl.BlockSpec((E, H), lambda: (0, 0)),
            pl.BlockSpec((E, 1), lambda: (0, 0)),
        ],
        out_specs=[
            pl.BlockSpec((E, N), lambda: (0, 0)),
            pl.BlockSpec((E, N), lambda: (0, 0)),
            pl.BlockSpec((N, H), lambda: (0, 0)),
        ],
        out_shape=[
            jax.ShapeDtypeStruct((E, N), jnp.float32),
            jax.ShapeDtypeStruct((E, N), jnp.float32),
            jax.ShapeDtypeStruct((N, H), jnp.bfloat16),
        ],
    )(x, Wg, biasT)

    mesh = plsc.VectorSubcoreMesh(core_axis_name="c", subcore_axis_name="s")
    cpw = E * TPW
    comb_flat = pl.kernel(
        _sc_routing_kernel,
        mesh=mesh,
        out_type=jax.ShapeDtypeStruct((E * N,), jnp.float32),
        scratch_types=[
            pltpu.VMEM((cpw,), jnp.float32),
            pltpu.VMEM((cpw,), jnp.float32),
            pltpu.VMEM((cpw,), jnp.float32),
        ],
    )(scm.reshape(-1), sbm.reshape(-1))
    combEN = comb_flat.reshape(E, N)

    out = pl.pallas_call(
        _moe_kernel,
        in_specs=[
            pl.BlockSpec((N, H), lambda: (0, 0)),
            pl.BlockSpec((E, N), lambda: (0, 0)),
            pl.BlockSpec(memory_space=pl.ANY),
            pl.BlockSpec(memory_space=pl.ANY),
            pl.BlockSpec(memory_space=pl.ANY),
            pl.BlockSpec(memory_space=pl.ANY),
            pl.BlockSpec(memory_space=pl.ANY),
            pl.BlockSpec(memory_space=pl.ANY),
        ],
        out_specs=pl.BlockSpec((N, H), lambda: (0, 0)),
        out_shape=jax.ShapeDtypeStruct((N, H), jnp.float32),
        scratch_shapes=[
            pltpu.VMEM((2, DFF, H), jnp.float32),
            pltpu.VMEM((2, DFF, H), jnp.float32),
            pltpu.VMEM((2, H, DFF), jnp.float32),
            pltpu.VMEM((N, E), jnp.float32),
            pltpu.VMEM((N, (E + 1) * DFF), jnp.bfloat16),
            pltpu.VMEM((H, (E + 1) * DFF), jnp.bfloat16),
            pltpu.SemaphoreType.DMA((2,)),
        ],
    )(xb, combEN, W_gate, W_up, W_down, Ws_gate, Ws_up, Ws_down)

    return out.reshape(orig_shape)


# speedup vs baseline: 1.0913x; 1.0913x over previous
import functools

import jax
import jax.numpy as jnp
from jax import lax
from jax.experimental import pallas as pl
from jax.experimental.pallas import tpu as pltpu
from jax.experimental.pallas import tpu_sc as plsc

E = 8
NGROUP = 4
GSIZE = E // NGROUP
H = 768
DFF = 384
SCALE = 2.5
N = 2048

NWORK = 32
TPW = N // NWORK

_RHS_T = (((1,), (1,)), ((), ()))


def _gate_prep_kernel(x_ref, wgate_ref, bias_ref, ssb_ref, xb_ref):
    x = x_ref[...]
    logits = lax.dot_general(
        x, wgate_ref[...], _RHS_T,
        precision=lax.Precision.DEFAULT,
        preferred_element_type=jnp.float32)
    st = jax.nn.sigmoid(logits.T)
    sb = st + bias_ref[...]
    stacked = jnp.concatenate([st, sb], axis=0)
    for w in range(NWORK):
        ssb_ref[w] = stacked[:, w * TPW:(w + 1) * TPW]
    xb_ref[...] = x.astype(jnp.bfloat16)


def _route16(sc, sb):
    neg = jnp.float32(-1e30)
    one = jnp.ones((16,), jnp.float32)
    zero = jnp.zeros((16,), jnp.float32)

    def f(b):
        return jnp.where(b, one, zero)

    g = [sb[2 * j] + sb[2 * j + 1] for j in range(NGROUP)]
    grank = []
    for j in range(NGROUP):
        rj = zero
        for k in range(NGROUP):
            if k == j:
                continue
            rj = rj + f(g[k] > g[j])
            if k < j:
                rj = rj + f(g[k] == g[j])
        grank.append(rj)
    masked = [jnp.where(grank[e // GSIZE] < 1.5, sb[e], neg) for e in range(E)]
    m1 = masked[0]
    for e in range(1, E):
        m1 = jnp.maximum(m1, masked[e])
    top1 = []
    seen = zero
    for e in range(E):
        hit = f(masked[e] == m1)
        t = hit * (one - seen)
        top1.append(t)
        seen = seen + t
    rest = [jnp.where(top1[e] > 0.5, neg, masked[e]) for e in range(E)]
    m2 = rest[0]
    for e in range(1, E):
        m2 = jnp.maximum(m2, rest[e])
    top2 = []
    seen = zero
    for e in range(E):
        hit = f(rest[e] == m2)
        t = hit * (one - seen)
        top2.append(t)
        seen = seen + t
    w1raw = zero
    w2raw = zero
    for e in range(E):
        w1raw = w1raw + top1[e] * sc[e]
        w2raw = w2raw + top2[e] * sc[e]
    inv = SCALE / (w1raw + w2raw + 1e-20)
    return [(top1[e] * w1raw + top2[e] * w2raw) * inv for e in range(E)]


def _sc_routing_kernel(ssb_hbm, comb_hbm, ssb_v, out_v):
    wid = lax.axis_index("s") * 2 + lax.axis_index("c")
    cpw = E * TPW
    pltpu.sync_copy(ssb_hbm.at[pl.ds(wid * 2 * cpw, 2 * cpw)], ssb_v)
    for t in range(TPW // 16):
        sc = [ssb_v[pl.ds(e * TPW + t * 16, 16)] for e in range(E)]
        sb = [ssb_v[pl.ds((E + e) * TPW + t * 16, 16)] for e in range(E)]
        comb = _route16(sc, sb)
        for e in range(E):
            out_v[pl.ds(e * TPW + t * 16, 16)] = comb[e]
    pltpu.sync_copy(out_v, comb_hbm.at[pl.ds(wid * cpw, cpw)])


def _moe_kernel(xb_ref, comb3_ref, wg_hbm, wu_hbm, wd_hbm,
                wsg_hbm, wsu_hbm, wsd_hbm, out_ref,
                g_stg, u_stg, d_stg, comb_ref, act_ref, wd2_ref, sems):
    def copies(e, b):
        if e < E:
            return (pltpu.make_async_copy(wg_hbm.at[e], g_stg.at[b], sems.at[b]),
                    pltpu.make_async_copy(wu_hbm.at[e], u_stg.at[b], sems.at[b]),
                    pltpu.make_async_copy(wd_hbm.at[e], d_stg.at[b], sems.at[b]))
        return (pltpu.make_async_copy(wsg_hbm, g_stg.at[b], sems.at[b]),
                pltpu.make_async_copy(wsu_hbm, u_stg.at[b], sems.at[b]),
                pltpu.make_async_copy(wsd_hbm, d_stg.at[b], sems.at[b]))

    for c in copies(0, 0):
        c.start()

    comb3 = comb3_ref[...]
    combT = jnp.concatenate([comb3[w] for w in range(NWORK)], axis=1)
    comb_ref[...] = combT.T
    for e in range(E + 1):
        b = e % 2
        if e < E:
            for c in copies(e + 1, 1 - b):
                c.start()
        for c in copies(e, b):
            c.wait()
        xb = xb_ref[...]
        wgb = g_stg[b].astype(jnp.bfloat16)
        wub = u_stg[b].astype(jnp.bfloat16)
        wd2_ref[:, e * DFF:(e + 1) * DFF] = d_stg[b].astype(jnp.bfloat16)
        gm = lax.dot_general(xb, wgb, _RHS_T,
                             preferred_element_type=jnp.float32)
        um = lax.dot_general(xb, wub, _RHS_T,
                             preferred_element_type=jnp.float32)
        act = gm * jax.nn.sigmoid(gm) * um
        if e < E:
            act = act * comb_ref[:, e:e + 1]
        act_ref[:, e * DFF:(e + 1) * DFF] = act.astype(jnp.bfloat16)
    out_ref[...] = lax.dot_general(act_ref[...], wd2_ref[...], _RHS_T,
                                   preferred_element_type=jnp.float32)


def kernel(hidden_states, image_mask, audio_mask, Wg, expert_bias,
           W_gate, W_up, W_down, Ws_gate, Ws_up, Ws_down):
    orig_shape = hidden_states.shape
    x = hidden_states.reshape(-1, H)
    biasT = expert_bias.reshape(E, 1)

    ssb, xb = pl.pallas_call(
        _gate_prep_kernel,
        in_specs=[
            pl.BlockSpec((N, H), lambda: (0, 0)),
            pl.BlockSpec((E, H), lambda: (0, 0)),
            pl.BlockSpec((E, 1), lambda: (0, 0)),
        ],
        out_specs=[
            pl.BlockSpec((NWORK, 2 * E, TPW), lambda: (0, 0, 0)),
            pl.BlockSpec((N, H), lambda: (0, 0)),
        ],
        out_shape=[
            jax.ShapeDtypeStruct((NWORK, 2 * E, TPW), jnp.float32),
            jax.ShapeDtypeStruct((N, H), jnp.bfloat16),
        ],
    )(x, Wg, biasT)

    mesh = plsc.VectorSubcoreMesh(core_axis_name="c", subcore_axis_name="s")
    cpw = E * TPW
    comb_flat = pl.kernel(
        _sc_routing_kernel,
        mesh=mesh,
        out_type=jax.ShapeDtypeStruct((NWORK * cpw,), jnp.float32),
        scratch_types=[
            pltpu.VMEM((2 * cpw,), jnp.float32),
            pltpu.VMEM((cpw,), jnp.float32),
        ],
    )(ssb.reshape(-1))
    comb3 = comb_flat.reshape(NWORK, E, TPW)

    out = pl.pallas_call(
        _moe_kernel,
        in_specs=[
            pl.BlockSpec((N, H), lambda: (0, 0)),
            pl.BlockSpec((NWORK, E, TPW), lambda: (0, 0, 0)),
            pl.BlockSpec(memory_space=pl.ANY),
            pl.BlockSpec(memory_space=pl.ANY),
            pl.BlockSpec(memory_space=pl.ANY),
            pl.BlockSpec(memory_space=pl.ANY),
            pl.BlockSpec(memory_space=pl.ANY),
            pl.BlockSpec(memory_space=pl.ANY),
        ],
        out_specs=pl.BlockSpec((N, H), lambda: (0, 0)),
        out_shape=jax.ShapeDtypeStruct((N, H), jnp.float32),
        scratch_shapes=[
            pltpu.VMEM((2, DFF, H), jnp.float32),
            pltpu.VMEM((2, DFF, H), jnp.float32),
            pltpu.VMEM((2, H, DFF), jnp.float32),
            pltpu.VMEM((N, E), jnp.float32),
            pltpu.VMEM((N, (E + 1) * DFF), jnp.bfloat16),
            pltpu.VMEM((H, (E + 1) * DFF), jnp.bfloat16),
            pltpu.SemaphoreType.DMA((2,)),
        ],
    )(xb, comb3, W_gate, W_up, W_down, Ws_gate, Ws_up, Ws_down)

    return out.reshape(orig_shape)


# speedup vs baseline: 1.1441x; 1.0484x over previous
import functools

import jax
import jax.numpy as jnp
from jax import lax
from jax.experimental import pallas as pl
from jax.experimental.pallas import tpu as pltpu
from jax.experimental.pallas import tpu_sc as plsc

E = 8
NGROUP = 4
GSIZE = E // NGROUP
H = 768
DFF = 384
SCALE = 2.5
N = 2048

NWORK = 16
TPW = N // NWORK

_RHS_T = (((1,), (1,)), ((), ()))


def _gate_prep_kernel(x_ref, wgate_ref, bias_ref, ssb_ref, xb_ref):
    x = x_ref[...]
    logits = lax.dot_general(
        x, wgate_ref[...], _RHS_T,
        precision=lax.Precision.DEFAULT,
        preferred_element_type=jnp.float32)
    st = jax.nn.sigmoid(logits.T)
    sb = st + bias_ref[...]
    stacked = jnp.concatenate([st, sb], axis=0)
    for w in range(NWORK):
        ssb_ref[w] = stacked[:, w * TPW:(w + 1) * TPW]
    xb_ref[...] = x.astype(jnp.bfloat16)


def _route16(sc, sb):
    neg = jnp.float32(-1e30)
    one = jnp.ones((16,), jnp.float32)
    zero = jnp.zeros((16,), jnp.float32)

    def f(b):
        return jnp.where(b, one, zero)

    g = [sb[2 * j] + sb[2 * j + 1] for j in range(NGROUP)]
    grank = []
    for j in range(NGROUP):
        rj = zero
        for k in range(NGROUP):
            if k == j:
                continue
            rj = rj + f(g[k] > g[j])
            if k < j:
                rj = rj + f(g[k] == g[j])
        grank.append(rj)
    masked = [jnp.where(grank[e // GSIZE] < 1.5, sb[e], neg) for e in range(E)]
    m1 = masked[0]
    for e in range(1, E):
        m1 = jnp.maximum(m1, masked[e])
    top1 = []
    seen = zero
    for e in range(E):
        hit = f(masked[e] == m1)
        t = hit * (one - seen)
        top1.append(t)
        seen = seen + t
    rest = [jnp.where(top1[e] > 0.5, neg, masked[e]) for e in range(E)]
    m2 = rest[0]
    for e in range(1, E):
        m2 = jnp.maximum(m2, rest[e])
    top2 = []
    seen = zero
    for e in range(E):
        hit = f(rest[e] == m2)
        t = hit * (one - seen)
        top2.append(t)
        seen = seen + t
    w1raw = zero
    w2raw = zero
    for e in range(E):
        w1raw = w1raw + top1[e] * sc[e]
        w2raw = w2raw + top2[e] * sc[e]
    inv = SCALE / (w1raw + w2raw + 1e-20)
    return [(top1[e] * w1raw + top2[e] * w2raw) * inv for e in range(E)]


def _sc_routing_kernel(ssb_hbm, comb_hbm, ssb_v, out_v):
    wid = lax.axis_index("s")
    cpw = E * TPW
    pltpu.sync_copy(ssb_hbm.at[pl.ds(wid * 2 * cpw, 2 * cpw)], ssb_v)
    for t in range(TPW // 16):
        sc = [ssb_v[pl.ds(e * TPW + t * 16, 16)] for e in range(E)]
        sb = [ssb_v[pl.ds((E + e) * TPW + t * 16, 16)] for e in range(E)]
        comb = _route16(sc, sb)
        for e in range(E):
            out_v[pl.ds(e * TPW + t * 16, 16)] = comb[e]
    pltpu.sync_copy(out_v, comb_hbm.at[pl.ds(wid * cpw, cpw)])


def _moe_kernel(xb_ref, comb3_ref, wg_hbm, wu_hbm, wd_hbm,
                wsg_hbm, wsu_hbm, wsd_hbm, out_ref,
                g_stg, u_stg, d_stg, comb_ref, act_ref, wd2_ref, sems):
    def copies(e, b):
        if e < E:
            return (pltpu.make_async_copy(wg_hbm.at[e], g_stg.at[b], sems.at[b]),
                    pltpu.make_async_copy(wu_hbm.at[e], u_stg.at[b], sems.at[b]),
                    pltpu.make_async_copy(wd_hbm.at[e], d_stg.at[b], sems.at[b]))
        return (pltpu.make_async_copy(wsg_hbm, g_stg.at[b], sems.at[b]),
                pltpu.make_async_copy(wsu_hbm, u_stg.at[b], sems.at[b]),
                pltpu.make_async_copy(wsd_hbm, d_stg.at[b], sems.at[b]))

    for c in copies(0, 0):
        c.start()

    comb3 = comb3_ref[...]
    combT = jnp.concatenate([comb3[w] for w in range(NWORK)], axis=1)
    comb_ref[...] = combT.T
    for e in range(E + 1):
        b = e % 2
        if e < E:
            for c in copies(e + 1, 1 - b):
                c.start()
        for c in copies(e, b):
            c.wait()
        xb = xb_ref[...]
        wgb = g_stg[b].astype(jnp.bfloat16)
        wub = u_stg[b].astype(jnp.bfloat16)
        wd2_ref[:, e * DFF:(e + 1) * DFF] = d_stg[b].astype(jnp.bfloat16)
        gm = lax.dot_general(xb, wgb, _RHS_T,
                             preferred_element_type=jnp.float32)
        um = lax.dot_general(xb, wub, _RHS_T,
                             preferred_element_type=jnp.float32)
        act = gm * jax.nn.sigmoid(gm) * um
        if e < E:
            act = act * comb_ref[:, e:e + 1]
        act_ref[:, e * DFF:(e + 1) * DFF] = act.astype(jnp.bfloat16)
    out_ref[...] = lax.dot_general(act_ref[...], wd2_ref[...], _RHS_T,
                                   preferred_element_type=jnp.float32)


def kernel(hidden_states, image_mask, audio_mask, Wg, expert_bias,
           W_gate, W_up, W_down, Ws_gate, Ws_up, Ws_down):
    orig_shape = hidden_states.shape
    x = hidden_states.reshape(-1, H)
    biasT = expert_bias.reshape(E, 1)

    ssb, xb = pl.pallas_call(
        _gate_prep_kernel,
        in_specs=[
            pl.BlockSpec((N, H), lambda: (0, 0)),
            pl.BlockSpec((E, H), lambda: (0, 0)),
            pl.BlockSpec((E, 1), lambda: (0, 0)),
        ],
        out_specs=[
            pl.BlockSpec((NWORK, 2 * E, TPW), lambda: (0, 0, 0)),
            pl.BlockSpec((N, H), lambda: (0, 0)),
        ],
        out_shape=[
            jax.ShapeDtypeStruct((NWORK, 2 * E, TPW), jnp.float32),
            jax.ShapeDtypeStruct((N, H), jnp.bfloat16),
        ],
    )(x, Wg, biasT)

    mesh = plsc.VectorSubcoreMesh(core_axis_name="c", subcore_axis_name="s",
                                  num_cores=1)
    cpw = E * TPW
    comb_flat = pl.kernel(
        _sc_routing_kernel,
        mesh=mesh,
        out_type=jax.ShapeDtypeStruct((NWORK * cpw,), jnp.float32),
        scratch_types=[
            pltpu.VMEM((2 * cpw,), jnp.float32),
            pltpu.VMEM((cpw,), jnp.float32),
        ],
    )(ssb.reshape(-1))
    comb3 = comb_flat.reshape(NWORK, E, TPW)

    out = pl.pallas_call(
        _moe_kernel,
        in_specs=[
            pl.BlockSpec((N, H), lambda: (0, 0)),
            pl.BlockSpec((NWORK, E, TPW), lambda: (0, 0, 0)),
            pl.BlockSpec(memory_space=pl.ANY),
            pl.BlockSpec(memory_space=pl.ANY),
            pl.BlockSpec(memory_space=pl.ANY),
            pl.BlockSpec(memory_space=pl.ANY),
            pl.BlockSpec(memory_space=pl.ANY),
            pl.BlockSpec(memory_space=pl.ANY),
        ],
        out_specs=pl.BlockSpec((N, H), lambda: (0, 0)),
        out_shape=jax.ShapeDtypeStruct((N, H), jnp.float32),
        scratch_shapes=[
            pltpu.VMEM((2, DFF, H), jnp.float32),
            pltpu.VMEM((2, DFF, H), jnp.float32),
            pltpu.VMEM((2, H, DFF), jnp.float32),
            pltpu.VMEM((N, E), jnp.float32),
            pltpu.VMEM((N, (E + 1) * DFF), jnp.bfloat16),
            pltpu.VMEM((H, (E + 1) * DFF), jnp.bfloat16),
            pltpu.SemaphoreType.DMA((2,)),
        ],
    )(xb, comb3, W_gate, W_up, W_down, Ws_gate, Ws_up, Ws_down)

    return out.reshape(orig_shape)
